# Initial kernel scaffold; baseline (speedup 1.0000x reference)
import functools

import jax
import jax.numpy as jnp
from jax import lax
from jax.experimental import pallas as pl
from jax.experimental.pallas import tpu as pltpu
from jax.experimental.pallas import tpu_sc as plsc

B = 256
NM = 1024
FEAT = 12288
KBLK = 1536
KSTEPS = FEAT // KBLK
GRID_N = 32

_NC, _NS = 2, 16
_NW = _NC * _NS
_B_PER_W = B // _NW


def _argmin_kernel(x_ref, w_ref, idx_ref, acc_ref):
    k = pl.program_id(0)
    part = lax.dot_general(
        x_ref[...], w_ref[...],
        dimension_numbers=(((1,), (1,)), ((), ())),
        preferred_element_type=jnp.float32,
    )
    wb = w_ref[...]
    w2 = jnp.sum(wb * wb, axis=1)
    upd = w2[None, :] - 2.0 * part

    @pl.when(k == 0)
    def _init():
        acc_ref[...] = upd

    @pl.when(k > 0)
    def _acc():
        acc_ref[...] += upd

    @pl.when(k == KSTEPS - 1)
    def _finish():
        scores = acc_ref[...]
        minv = jnp.min(scores, axis=1, keepdims=True)
        iota = lax.broadcasted_iota(jnp.int32, scores.shape, 1)
        idx = jnp.min(jnp.where(scores == minv, iota, NM), axis=1,
                      keepdims=True)
        flat = (idx % GRID_N) * GRID_N + idx // GRID_N
        idx_ref[...] = flat


def _compute_indices(xf, wf):
    return pl.pallas_call(
        _argmin_kernel,
        grid=(KSTEPS,),
        in_specs=[
            pl.BlockSpec((B, KBLK), lambda k: (0, k)),
            pl.BlockSpec((NM, KBLK), lambda k: (0, k)),
        ],
        out_specs=pl.BlockSpec((B, 1), lambda k: (0, 0)),
        out_shape=jax.ShapeDtypeStruct((B, 1), jnp.int32),
        scratch_shapes=[pltpu.VMEM((B, NM), jnp.float32)],
    )(xf, wf)


def _sc_gather(wf, idx):
    mesh = plsc.VectorSubcoreMesh(core_axis_name="c", subcore_axis_name="s")

    @functools.partial(
        pl.kernel,
        mesh=mesh,
        out_type=jax.ShapeDtypeStruct((B, FEAT), jnp.float32),
        scratch_types=[
            pltpu.VMEM((_B_PER_W,), jnp.int32),
            pltpu.VMEM((_B_PER_W, FEAT), jnp.float32),
            pltpu.SemaphoreType.DMA,
        ],
    )
    def gather_kernel(table_hbm, idx_hbm, out_hbm, idx_v, rows_v, sem):
        wid = lax.axis_index("s") * _NC + lax.axis_index("c")
        base = wid * _B_PER_W
        pltpu.sync_copy(idx_hbm.at[pl.ds(base, _B_PER_W)], idx_v)
        pltpu.async_copy(table_hbm.at[idx_v], rows_v, sem).wait()
        pltpu.sync_copy(rows_v, out_hbm.at[pl.ds(base, _B_PER_W)])

    return gather_kernel(wf, idx)


def kernel(x, weights):
    xf = x.reshape(B, FEAT)
    wf = weights.reshape(NM, FEAT)
    idx = _compute_indices(xf, wf).reshape(B)
    rows = _sc_gather(wf, idx)
    return rows.reshape(B, *weights.shape[2:])


# speedup vs baseline: 1.0500x; 1.0500x over previous
import functools

import jax
import jax.numpy as jnp
from jax import lax
from jax.experimental import pallas as pl
from jax.experimental.pallas import tpu as pltpu
from jax.experimental.pallas import tpu_sc as plsc

B = 256
NM = 1024
FEAT = 12288
KBLK = 1536
KSTEPS = FEAT // KBLK
GRID_N = 32

_NC, _NS = 2, 16
_NW = _NC * _NS
_B_PER_W = B // _NW


def _argmin_kernel(x_ref, w_ref, idx_ref, acc_ref):
    k = pl.program_id(0)
    part = lax.dot_general(
        w_ref[...], x_ref[...],
        dimension_numbers=(((1,), (1,)), ((), ())),
        preferred_element_type=jnp.float32,
    )
    w2 = jnp.zeros((NM, 1), jnp.float32)
    for c in range(KBLK // 128):
        blk = w_ref[:, c * 128:(c + 1) * 128]
        w2 = w2 + jnp.sum(blk * blk, axis=1, keepdims=True)
    upd = w2 - 2.0 * part

    @pl.when(k == 0)
    def _init():
        acc_ref[...] = upd

    @pl.when(k > 0)
    def _acc():
        acc_ref[...] += upd

    @pl.when(k == KSTEPS - 1)
    def _finish():
        scores = acc_ref[...]
        minv = jnp.min(scores, axis=0, keepdims=True)
        iota = lax.broadcasted_iota(jnp.int32, scores.shape, 0)
        idx = jnp.min(jnp.where(scores == minv, iota, NM), axis=0,
                      keepdims=True)
        flat = (idx % GRID_N) * GRID_N + idx // GRID_N
        idx_ref[...] = flat


def _compute_indices(xf, wf):
    return pl.pallas_call(
        _argmin_kernel,
        grid=(KSTEPS,),
        in_specs=[
            pl.BlockSpec((B, KBLK), lambda k: (0, k)),
            pl.BlockSpec((NM, KBLK), lambda k: (0, k)),
        ],
        out_specs=pl.BlockSpec((1, B), lambda k: (0, 0)),
        out_shape=jax.ShapeDtypeStruct((1, B), jnp.int32),
        scratch_shapes=[pltpu.VMEM((NM, B), jnp.float32)],
    )(xf, wf)


def _sc_gather(wf, idx):
    mesh = plsc.VectorSubcoreMesh(core_axis_name="c", subcore_axis_name="s")

    @functools.partial(
        pl.kernel,
        mesh=mesh,
        out_type=jax.ShapeDtypeStruct((B, FEAT), jnp.float32),
        scratch_types=[
            pltpu.VMEM((_B_PER_W,), jnp.int32),
            pltpu.VMEM((_B_PER_W, FEAT), jnp.float32),
            pltpu.SemaphoreType.DMA,
        ],
    )
    def gather_kernel(table_hbm, idx_hbm, out_hbm, idx_v, rows_v, sem):
        wid = lax.axis_index("s") * _NC + lax.axis_index("c")
        base = wid * _B_PER_W
        pltpu.sync_copy(idx_hbm.at[pl.ds(base, _B_PER_W)], idx_v)
        pltpu.async_copy(table_hbm.at[idx_v], rows_v, sem).wait()
        pltpu.sync_copy(rows_v, out_hbm.at[pl.ds(base, _B_PER_W)])

    return gather_kernel(wf, idx)


def kernel(x, weights):
    xf = x.reshape(B, FEAT)
    wf = weights.reshape(NM, FEAT)
    idx = _compute_indices(xf, wf).reshape(B)
    rows = _sc_gather(wf, idx)
    return rows.reshape(B, *weights.shape[2:])


# speedup vs baseline: 1.2113x; 1.1536x over previous
import jax
import jax.numpy as jnp
from jax import lax
from jax.experimental import pallas as pl
from jax.experimental.pallas import tpu as pltpu

B = 256
NM = 1024
FEAT = 12288
KBLK = 1536
KSTEPS = FEAT // KBLK
FBLK = 1024
FSTEPS = FEAT // FBLK
GRID_N = 32


def _argmin_kernel(xt_ref, wt_ref, idx_ref, acc_ref):
    k = pl.program_id(0)
    part = lax.dot_general(
        xt_ref[...], wt_ref[...],
        dimension_numbers=(((0,), (0,)), ((), ())),
        preferred_element_type=jnp.float32,
    )
    w2 = jnp.zeros((1, NM), jnp.float32)
    for c in range(KBLK // 128):
        blk = wt_ref[c * 128:(c + 1) * 128, :]
        w2 = w2 + jnp.sum(blk * blk, axis=0, keepdims=True)
    upd = w2 - 2.0 * part

    @pl.when(k == 0)
    def _init():
        acc_ref[...] = upd

    @pl.when(k > 0)
    def _acc():
        acc_ref[...] += upd

    @pl.when(k == KSTEPS - 1)
    def _finish():
        scores = acc_ref[...]
        minv = jnp.min(scores, axis=1, keepdims=True)
        iota = lax.broadcasted_iota(jnp.int32, scores.shape, 1)
        idx = jnp.min(jnp.where(scores == minv, iota, NM), axis=1,
                      keepdims=True)
        flat = (idx % GRID_N) * GRID_N + idx // GRID_N
        idx_ref[...] = flat


def _compute_indices(xt, wt):
    return pl.pallas_call(
        _argmin_kernel,
        grid=(KSTEPS,),
        in_specs=[
            pl.BlockSpec((KBLK, B), lambda k: (k, 0)),
            pl.BlockSpec((KBLK, NM), lambda k: (k, 0)),
        ],
        out_specs=pl.BlockSpec((B, 1), lambda k: (0, 0)),
        out_shape=jax.ShapeDtypeStruct((B, 1), jnp.int32),
        scratch_shapes=[pltpu.VMEM((B, NM), jnp.float32)],
    )(xt, wt)


def _gather_kernel(wt_ref, idx_ref, out_ref):
    onehot = jnp.where(
        lax.broadcasted_iota(jnp.int32, (NM, B), 0) == idx_ref[...],
        1.0, 0.0).astype(jnp.float32)
    out_ref[...] = lax.dot_general(
        wt_ref[...], onehot,
        dimension_numbers=(((1,), (0,)), ((), ())),
        preferred_element_type=jnp.float32,
    )


def _gather_rows(wt, idx_row):
    return pl.pallas_call(
        _gather_kernel,
        grid=(FSTEPS,),
        in_specs=[
            pl.BlockSpec((FBLK, NM), lambda k: (k, 0)),
            pl.BlockSpec((1, B), lambda k: (0, 0)),
        ],
        out_specs=pl.BlockSpec((FBLK, B), lambda k: (k, 0)),
        out_shape=jax.ShapeDtypeStruct((FEAT, B), jnp.float32),
    )(wt, idx_row)


def kernel(x, weights):
    xt = jnp.transpose(x.reshape(B, FEAT))
    wt = jnp.transpose(weights.reshape(NM, FEAT))
    idx = _compute_indices(xt, wt)
    out_fm = _gather_rows(wt, idx.reshape(1, B))
    return jnp.transpose(out_fm).reshape(B, 64, 64, 3)
